# baseline (speedup 1.0000x reference)
import jax
import jax.numpy as jnp
from jax.experimental import pallas as pl
from jax.experimental.pallas import tpu as pltpu

_EPS = 1e-12
_FC = 4


def _embed_ln_kernel(grid_ref, row_ref, col_ref, tt_ref, gamma_ref, beta_ref,
                     out_ref, acc_ref):
    f = pl.program_id(1)
    nchunks = pl.num_programs(1)
    s = jnp.sum(grid_ref[0], axis=0)

    @pl.when(f == 0)
    def _init():
        acc_ref[...] = s

    @pl.when(f != 0)
    def _acc():
        acc_ref[...] += s

    @pl.when(f == nchunks - 1)
    def _finish():
        x = acc_ref[...] + row_ref[...][:, None, :] + col_ref[...][None, :, :]
        x = x + tt_ref[...][None, :, :]
        mu = jnp.mean(x, axis=-1, keepdims=True)
        var = jnp.mean(jnp.square(x - mu), axis=-1, keepdims=True)
        xhat = (x - mu) * jax.lax.rsqrt(var + _EPS)
        y = xhat * gamma_ref[...][None, :, :] + beta_ref[...][None, :, :]
        out_ref[0] = y.reshape(out_ref.shape[1], out_ref.shape[2])


def kernel(grid, row_table, col_table, tt_table, gamma, beta):
    B, NFRM, H, W, D = grid.shape
    scale = jnp.float32(NFRM)
    row_s = row_table[:H] * scale
    col_s = col_table[:W] * scale
    tt_s = tt_table * scale
    gamma2 = gamma.reshape(1, D)
    beta2 = beta.reshape(1, D)
    out = pl.pallas_call(
        _embed_ln_kernel,
        grid=(B, NFRM // _FC),
        in_specs=[
            pl.BlockSpec((1, _FC, H, W, D), lambda b, f: (b, f, 0, 0, 0)),
            pl.BlockSpec((H, D), lambda b, f: (0, 0)),
            pl.BlockSpec((W, D), lambda b, f: (0, 0)),
            pl.BlockSpec((1, D), lambda b, f: (0, 0)),
            pl.BlockSpec((1, D), lambda b, f: (0, 0)),
            pl.BlockSpec((1, D), lambda b, f: (0, 0)),
        ],
        out_specs=pl.BlockSpec((1, H * W, D), lambda b, f: (b, 0, 0)),
        out_shape=jax.ShapeDtypeStruct((B, H * W, D), grid.dtype),
        scratch_shapes=[pltpu.VMEM((H, W, D), jnp.float32)],
        compiler_params=pltpu.CompilerParams(
            dimension_semantics=("parallel", "arbitrary"),
        ),
    )(grid, row_s, col_s, tt_s, gamma2, beta2)
    return out
